# baseline (speedup 1.0000x reference)
import functools

import jax
import jax.numpy as jnp
from jax import lax
from jax.experimental import pallas as pl
from jax.experimental.pallas import tpu as pltpu
from jax.experimental.pallas import tpu_sc as plsc

BATCH = 16384
L = 200
LP = 208
NUM_CORES = 2
NUM_SUBCORES = 16
NW = NUM_CORES * NUM_SUBCORES
ROWS_PER_W = BATCH // NW
DMA_ROWS = 64
NV = DMA_ROWS // 16
GROUPS_PER_W = ROWS_PER_W // DMA_ROWS
CHUNK = DMA_ROWS * LP
P_HBM = 64
H_WORDS = P_HBM * DMA_ROWS
S_WORDS = CHUNK - H_WORDS
VOCAB = 1000001
TBL_ALLOC = 1000448
TBL_CHUNK = 62504
TBL_SUB = 13312
TBL_LAST = TBL_CHUNK - 4 * TBL_SUB
TBL_LAST15 = VOCAB - 15 * TBL_CHUNK - 4 * TBL_SUB


def _sc_embed_sum(x_flat, s_flat, wv, bv, tv):
    mesh = plsc.VectorSubcoreMesh(core_axis_name="c", subcore_axis_name="s")

    @functools.partial(
        pl.kernel,
        mesh=mesh,
        out_type=[
            jax.ShapeDtypeStruct((BATCH,), jnp.float32),
            jax.ShapeDtypeStruct((BATCH,), jnp.int32),
        ],
        scratch_types=[
            pltpu.VMEM_SHARED((TBL_ALLOC,), jnp.float32),
            pltpu.VMEM((CHUNK,), jnp.int32),
            pltpu.VMEM((CHUNK,), jnp.int32),
            pltpu.VMEM((CHUNK,), jnp.float32),
            pltpu.VMEM((CHUNK,), jnp.float32),
            pltpu.VMEM((ROWS_PER_W,), jnp.float32),
            pltpu.VMEM((ROWS_PER_W,), jnp.int32),
            pltpu.VMEM((16,), jnp.float32),
            pltpu.VMEM((16,), jnp.float32),
            pltpu.VMEM((16,), jnp.float32),
            pltpu.SemaphoreType.DMA,
            pltpu.SemaphoreType.DMA,
            pltpu.SemaphoreType.DMA,
            pltpu.SemaphoreType.DMA,
            pltpu.SemaphoreType.DMA,
            pltpu.SemaphoreType.DMA,
        ],
    )
    def k(x_hbm, s_hbm, wv_hbm, bv_hbm, tv_hbm, logit_hbm, label_hbm,
          table_sh, idx0, idx1, vals0, vals1, acc_v, lbl_v,
          wv_v, bv_v, tv_v, sem_i0, sem_i1, sem_v0, sem_v1, sem_h0, sem_h1):
        cid = lax.axis_index("c")
        sid = lax.axis_index("s")
        wid = sid * NUM_CORES + cid

        tbl_base = sid * TBL_CHUNK
        bufs = [vals0, vals1]
        sin = [sem_i0, sem_i1]
        sout = [sem_v0, sem_v1]
        last = jnp.where(sid == NUM_SUBCORES - 1, TBL_LAST15, TBL_LAST)

        def st_in(j, sz):
            pltpu.async_copy(s_hbm.at[pl.ds(tbl_base + j * TBL_SUB, sz)],
                             bufs[j % 2].at[pl.ds(0, sz)], sin[j % 2])

        def st_wait_in(j, sz):
            pltpu.make_async_copy(s_hbm.at[pl.ds(0, sz)],
                                  bufs[j % 2].at[pl.ds(0, sz)],
                                  sin[j % 2]).wait()

        def st_out(j, sz):
            pltpu.async_copy(bufs[j % 2].at[pl.ds(0, sz)],
                             table_sh.at[pl.ds(tbl_base + j * TBL_SUB, sz)],
                             sout[j % 2])

        def st_wait_out(j, sz):
            pltpu.make_async_copy(bufs[j % 2].at[pl.ds(0, sz)],
                                  table_sh.at[pl.ds(0, sz)],
                                  sout[j % 2]).wait()

        st_in(0, TBL_SUB)
        st_in(1, TBL_SUB)
        for j in range(4):
            st_wait_in(j, TBL_SUB)
            st_out(j, TBL_SUB)
            if j + 2 < 4:
                st_wait_out(j, TBL_SUB)
                st_in(j + 2, TBL_SUB)
        st_wait_out(2, TBL_SUB)

        @pl.when(sid == NUM_SUBCORES - 1)
        def _():
            pltpu.sync_copy(s_hbm.at[pl.ds(tbl_base + 4 * TBL_SUB, TBL_LAST15)],
                            vals0.at[pl.ds(0, TBL_LAST15)])
            pltpu.sync_copy(vals0.at[pl.ds(0, TBL_LAST15)],
                            table_sh.at[pl.ds(tbl_base + 4 * TBL_SUB, TBL_LAST15)])

        @pl.when(sid != NUM_SUBCORES - 1)
        def _():
            pltpu.sync_copy(s_hbm.at[pl.ds(tbl_base + 4 * TBL_SUB, TBL_LAST)],
                            vals0.at[pl.ds(0, TBL_LAST)])
            pltpu.sync_copy(vals0.at[pl.ds(0, TBL_LAST)],
                            table_sh.at[pl.ds(tbl_base + 4 * TBL_SUB, TBL_LAST)])

        st_wait_out(3, TBL_SUB)
        pltpu.sync_copy(wv_hbm, wv_v)
        pltpu.sync_copy(bv_hbm, bv_v)
        pltpu.sync_copy(tv_hbm, tv_v)
        w = wv_v[...]
        b = bv_v[...]
        t = tv_v[...]
        plsc.subcore_barrier()

        def issue_idx(g, idx_buf, sem):
            gc = jnp.minimum(g, GROUPS_PER_W - 1)
            base = (wid * GROUPS_PER_W + gc) * CHUNK
            pltpu.async_copy(x_hbm.at[pl.ds(base, CHUNK)], idx_buf, sem)

        def wait_idx(idx_buf, sem):
            pltpu.make_async_copy(x_hbm.at[pl.ds(0, CHUNK)], idx_buf, sem).wait()

        def issue_gather(idx_buf, vals_buf, sem, sem_h):
            pltpu.async_copy(s_hbm.at[idx_buf.at[pl.ds(0, H_WORDS)]],
                             vals_buf.at[pl.ds(0, H_WORDS)], sem_h)
            pltpu.async_copy(table_sh.at[idx_buf.at[pl.ds(H_WORDS, S_WORDS)]],
                             vals_buf.at[pl.ds(H_WORDS, S_WORDS)], sem)

        def wait_gather(idx_buf, vals_buf, sem, sem_h):
            pltpu.make_async_copy(
                s_hbm.at[idx_buf.at[pl.ds(0, H_WORDS)]],
                vals_buf.at[pl.ds(0, H_WORDS)], sem_h).wait()
            pltpu.make_async_copy(
                table_sh.at[idx_buf.at[pl.ds(H_WORDS, S_WORDS)]],
                vals_buf.at[pl.ds(H_WORDS, S_WORDS)], sem).wait()

        def compute(g, vals_buf):
            def p_body(p, accs):
                return tuple(
                    accs[v] + vals_buf[pl.ds((p * NV + v) * 16, 16)]
                    for v in range(NV)
                )

            accs = lax.fori_loop(
                0, LP, p_body,
                tuple(jnp.zeros((16,), jnp.float32) for _ in range(NV)))
            for v in range(NV):
                logit = accs[v] * w + b
                label = jnp.where(logit >= t, 1, 0).astype(jnp.int32)
                acc_v[pl.ds(g * DMA_ROWS + v * 16, 16)] = logit
                lbl_v[pl.ds(g * DMA_ROWS + v * 16, 16)] = label

        pltpu.sync_copy(x_hbm.at[pl.ds(wid * GROUPS_PER_W * CHUNK, CHUNK)], idx0)
        issue_gather(idx0, vals0, sem_v0, sem_h0)
        issue_idx(1, idx1, sem_i1)

        def pair_body(i, carry):
            g0 = 2 * i
            g1 = g0 + 1
            wait_gather(idx0, vals0, sem_v0, sem_h0)
            issue_idx(g0 + 2, idx0, sem_i0)
            wait_idx(idx1, sem_i1)
            issue_gather(idx1, vals1, sem_v1, sem_h1)
            compute(g0, vals0)
            wait_gather(idx1, vals1, sem_v1, sem_h1)
            issue_idx(g1 + 2, idx1, sem_i1)
            wait_idx(idx0, sem_i0)
            issue_gather(idx0, vals0, sem_v0, sem_h0)
            compute(g1, vals1)
            return carry

        lax.fori_loop(0, GROUPS_PER_W // 2, pair_body, 0)
        wait_gather(idx0, vals0, sem_v0, sem_h0)
        wait_idx(idx1, sem_i1)

        out_base = wid * ROWS_PER_W
        pltpu.sync_copy(acc_v, logit_hbm.at[pl.ds(out_base, ROWS_PER_W)])
        pltpu.sync_copy(lbl_v, label_hbm.at[pl.ds(out_base, ROWS_PER_W)])

    return k(x_flat, s_flat, wv, bv, tv)


def kernel(x, S, ones_col, W, b, thresh_t):
    xp = jnp.pad(x.astype(jnp.int32), ((0, 0), (0, LP - L)))
    x_flat = xp.reshape(BATCH // DMA_ROWS, DMA_ROWS, LP)
    x_flat = x_flat.transpose(0, 2, 1).reshape(-1)
    s_flat = S.reshape(-1)
    wv = jnp.broadcast_to(W.reshape(1), (16,))
    bv = jnp.broadcast_to(b.reshape(1), (16,))
    tv = jnp.broadcast_to(thresh_t.reshape(1), (16,))
    logit, label = _sc_embed_sum(x_flat, s_flat, wv, bv, tv)
    return (logit.reshape(BATCH, 1), label.astype(jnp.bool_).reshape(BATCH, 1))


# speedup vs baseline: 1.0056x; 1.0056x over previous
import functools

import jax
import jax.numpy as jnp
from jax import lax
from jax.experimental import pallas as pl
from jax.experimental.pallas import tpu as pltpu
from jax.experimental.pallas import tpu_sc as plsc

BATCH = 16384
L = 200
LP = 208
NUM_CORES = 2
NUM_SUBCORES = 16
NW = NUM_CORES * NUM_SUBCORES
ROWS_PER_W = BATCH // NW
DMA_ROWS = 64
NV = DMA_ROWS // 16
GROUPS_PER_W = ROWS_PER_W // DMA_ROWS
CHUNK = DMA_ROWS * LP
P_HBM = 72
H_WORDS = P_HBM * DMA_ROWS
S_WORDS = CHUNK - H_WORDS
VOCAB = 1000001
TBL_ALLOC = 1000448
TBL_CHUNK = 62504
TBL_SUB = 13312
TBL_LAST = TBL_CHUNK - 4 * TBL_SUB
TBL_LAST15 = VOCAB - 15 * TBL_CHUNK - 4 * TBL_SUB


def _sc_embed_sum(x_flat, s_flat, wv, bv, tv):
    mesh = plsc.VectorSubcoreMesh(core_axis_name="c", subcore_axis_name="s")

    @functools.partial(
        pl.kernel,
        mesh=mesh,
        out_type=[
            jax.ShapeDtypeStruct((BATCH,), jnp.float32),
            jax.ShapeDtypeStruct((BATCH,), jnp.int32),
        ],
        scratch_types=[
            pltpu.VMEM_SHARED((TBL_ALLOC,), jnp.float32),
            pltpu.VMEM((CHUNK,), jnp.int32),
            pltpu.VMEM((CHUNK,), jnp.int32),
            pltpu.VMEM((CHUNK,), jnp.float32),
            pltpu.VMEM((CHUNK,), jnp.float32),
            pltpu.VMEM((ROWS_PER_W,), jnp.float32),
            pltpu.VMEM((ROWS_PER_W,), jnp.int32),
            pltpu.VMEM((16,), jnp.float32),
            pltpu.VMEM((16,), jnp.float32),
            pltpu.VMEM((16,), jnp.float32),
            pltpu.SemaphoreType.DMA,
            pltpu.SemaphoreType.DMA,
            pltpu.SemaphoreType.DMA,
            pltpu.SemaphoreType.DMA,
            pltpu.SemaphoreType.DMA,
            pltpu.SemaphoreType.DMA,
        ],
    )
    def k(x_hbm, s_hbm, wv_hbm, bv_hbm, tv_hbm, logit_hbm, label_hbm,
          table_sh, idx0, idx1, vals0, vals1, acc_v, lbl_v,
          wv_v, bv_v, tv_v, sem_i0, sem_i1, sem_v0, sem_v1, sem_h0, sem_h1):
        cid = lax.axis_index("c")
        sid = lax.axis_index("s")
        wid = sid * NUM_CORES + cid

        tbl_base = sid * TBL_CHUNK
        bufs = [vals0, vals1]
        sin = [sem_i0, sem_i1]
        sout = [sem_v0, sem_v1]
        last = jnp.where(sid == NUM_SUBCORES - 1, TBL_LAST15, TBL_LAST)

        def st_in(j, sz):
            pltpu.async_copy(s_hbm.at[pl.ds(tbl_base + j * TBL_SUB, sz)],
                             bufs[j % 2].at[pl.ds(0, sz)], sin[j % 2])

        def st_wait_in(j, sz):
            pltpu.make_async_copy(s_hbm.at[pl.ds(0, sz)],
                                  bufs[j % 2].at[pl.ds(0, sz)],
                                  sin[j % 2]).wait()

        def st_out(j, sz):
            pltpu.async_copy(bufs[j % 2].at[pl.ds(0, sz)],
                             table_sh.at[pl.ds(tbl_base + j * TBL_SUB, sz)],
                             sout[j % 2])

        def st_wait_out(j, sz):
            pltpu.make_async_copy(bufs[j % 2].at[pl.ds(0, sz)],
                                  table_sh.at[pl.ds(0, sz)],
                                  sout[j % 2]).wait()

        st_in(0, TBL_SUB)
        st_in(1, TBL_SUB)
        for j in range(4):
            st_wait_in(j, TBL_SUB)
            st_out(j, TBL_SUB)
            if j + 2 < 4:
                st_wait_out(j, TBL_SUB)
                st_in(j + 2, TBL_SUB)
        st_wait_out(2, TBL_SUB)

        @pl.when(sid == NUM_SUBCORES - 1)
        def _():
            pltpu.sync_copy(s_hbm.at[pl.ds(tbl_base + 4 * TBL_SUB, TBL_LAST15)],
                            vals0.at[pl.ds(0, TBL_LAST15)])
            pltpu.sync_copy(vals0.at[pl.ds(0, TBL_LAST15)],
                            table_sh.at[pl.ds(tbl_base + 4 * TBL_SUB, TBL_LAST15)])

        @pl.when(sid != NUM_SUBCORES - 1)
        def _():
            pltpu.sync_copy(s_hbm.at[pl.ds(tbl_base + 4 * TBL_SUB, TBL_LAST)],
                            vals0.at[pl.ds(0, TBL_LAST)])
            pltpu.sync_copy(vals0.at[pl.ds(0, TBL_LAST)],
                            table_sh.at[pl.ds(tbl_base + 4 * TBL_SUB, TBL_LAST)])

        st_wait_out(3, TBL_SUB)
        pltpu.sync_copy(wv_hbm, wv_v)
        pltpu.sync_copy(bv_hbm, bv_v)
        pltpu.sync_copy(tv_hbm, tv_v)
        w = wv_v[...]
        b = bv_v[...]
        t = tv_v[...]
        plsc.subcore_barrier()

        def issue_idx(g, idx_buf, sem):
            gc = jnp.minimum(g, GROUPS_PER_W - 1)
            base = (wid * GROUPS_PER_W + gc) * CHUNK
            pltpu.async_copy(x_hbm.at[pl.ds(base, CHUNK)], idx_buf, sem)

        def wait_idx(idx_buf, sem):
            pltpu.make_async_copy(x_hbm.at[pl.ds(0, CHUNK)], idx_buf, sem).wait()

        def issue_gather(idx_buf, vals_buf, sem, sem_h):
            pltpu.async_copy(s_hbm.at[idx_buf.at[pl.ds(0, H_WORDS)]],
                             vals_buf.at[pl.ds(0, H_WORDS)], sem_h)
            pltpu.async_copy(table_sh.at[idx_buf.at[pl.ds(H_WORDS, S_WORDS)]],
                             vals_buf.at[pl.ds(H_WORDS, S_WORDS)], sem)

        def wait_gather(idx_buf, vals_buf, sem, sem_h):
            pltpu.make_async_copy(
                s_hbm.at[idx_buf.at[pl.ds(0, H_WORDS)]],
                vals_buf.at[pl.ds(0, H_WORDS)], sem_h).wait()
            pltpu.make_async_copy(
                table_sh.at[idx_buf.at[pl.ds(H_WORDS, S_WORDS)]],
                vals_buf.at[pl.ds(H_WORDS, S_WORDS)], sem).wait()

        def compute(g, vals_buf):
            def p_body(p, accs):
                return tuple(
                    accs[v] + vals_buf[pl.ds((p * NV + v) * 16, 16)]
                    for v in range(NV)
                )

            accs = lax.fori_loop(
                0, LP, p_body,
                tuple(jnp.zeros((16,), jnp.float32) for _ in range(NV)))
            for v in range(NV):
                logit = accs[v] * w + b
                label = jnp.where(logit >= t, 1, 0).astype(jnp.int32)
                acc_v[pl.ds(g * DMA_ROWS + v * 16, 16)] = logit
                lbl_v[pl.ds(g * DMA_ROWS + v * 16, 16)] = label

        pltpu.sync_copy(x_hbm.at[pl.ds(wid * GROUPS_PER_W * CHUNK, CHUNK)], idx0)
        issue_gather(idx0, vals0, sem_v0, sem_h0)
        issue_idx(1, idx1, sem_i1)

        def pair_body(i, carry):
            g0 = 2 * i
            g1 = g0 + 1
            wait_gather(idx0, vals0, sem_v0, sem_h0)
            issue_idx(g0 + 2, idx0, sem_i0)
            wait_idx(idx1, sem_i1)
            issue_gather(idx1, vals1, sem_v1, sem_h1)
            compute(g0, vals0)
            wait_gather(idx1, vals1, sem_v1, sem_h1)
            issue_idx(g1 + 2, idx1, sem_i1)
            wait_idx(idx0, sem_i0)
            issue_gather(idx0, vals0, sem_v0, sem_h0)
            compute(g1, vals1)
            return carry

        lax.fori_loop(0, GROUPS_PER_W // 2, pair_body, 0)
        wait_gather(idx0, vals0, sem_v0, sem_h0)
        wait_idx(idx1, sem_i1)

        out_base = wid * ROWS_PER_W
        pltpu.sync_copy(acc_v, logit_hbm.at[pl.ds(out_base, ROWS_PER_W)])
        pltpu.sync_copy(lbl_v, label_hbm.at[pl.ds(out_base, ROWS_PER_W)])

    return k(x_flat, s_flat, wv, bv, tv)


def kernel(x, S, ones_col, W, b, thresh_t):
    xp = jnp.pad(x.astype(jnp.int32), ((0, 0), (0, LP - L)))
    x_flat = xp.reshape(BATCH // DMA_ROWS, DMA_ROWS, LP)
    x_flat = x_flat.transpose(0, 2, 1).reshape(-1)
    s_flat = S.reshape(-1)
    wv = jnp.broadcast_to(W.reshape(1), (16,))
    bv = jnp.broadcast_to(b.reshape(1), (16,))
    tv = jnp.broadcast_to(thresh_t.reshape(1), (16,))
    logit, label = _sc_embed_sum(x_flat, s_flat, wv, bv, tv)
    return (logit.reshape(BATCH, 1), label.astype(jnp.bool_).reshape(BATCH, 1))
